# baseline (speedup 1.0000x reference)
import functools

import jax
import jax.numpy as jnp
from jax import lax
from jax.experimental import pallas as pl
from jax.experimental.pallas import tpu as pltpu
from jax.experimental.pallas import tpu_sc as plsc

N_NODES = 10000
N_EDGES = 320000
D = 128

NC = 2
NS = 16
NW = NC * NS
CH = 128
K = 80
DCH = 128
DK = 80
E_PAD = NW * K * CH
N_DUMP = 1000
N_PAD = N_NODES + N_DUMP

ROWS_PER_SUB = (N_PAD // NS) // 8 * 8
TAIL_ROWS = N_PAD - NS * ROWS_PER_SUB

_mesh = plsc.VectorSubcoreMesh(core_axis_name="c", subcore_axis_name="s")


@functools.partial(
    pl.kernel,
    out_type=jax.ShapeDtypeStruct((NC, N_PAD, D), jnp.float32),
    mesh=_mesh,
    scratch_types=[
        pltpu.VMEM((DK, DCH), jnp.int32),
        pltpu.VMEM((DCH, D), jnp.float32),
        pltpu.VMEM_SHARED((N_PAD, D), jnp.float32),
        pltpu.SemaphoreType.DMA,
    ],
)
def _deg_kernel(src_hbm, zeros_hbm, ones_hbm, out_hbm, idx_v, ones_v, deg_sh,
                sem):
    c = lax.axis_index("c")
    s = lax.axis_index("s")
    wid = s * NC + c
    base = s * ROWS_PER_SUB
    pltpu.sync_copy(zeros_hbm.at[pl.ds(base, ROWS_PER_SUB)],
                    deg_sh.at[pl.ds(base, ROWS_PER_SUB)])

    @pl.when(s == NS - 1)
    def _():
        pltpu.sync_copy(zeros_hbm.at[pl.ds(NS * ROWS_PER_SUB, TAIL_ROWS)],
                        deg_sh.at[pl.ds(NS * ROWS_PER_SUB, TAIL_ROWS)])

    pltpu.sync_copy(ones_hbm, ones_v)
    pltpu.sync_copy(src_hbm.at[wid], idx_v)
    plsc.subcore_barrier()

    @pl.loop(0, DK, step=4)
    def _(j):
        h0 = pltpu.async_copy(ones_v, deg_sh.at[idx_v.at[j]], sem, add=True)
        h1 = pltpu.async_copy(ones_v, deg_sh.at[idx_v.at[j + 1]], sem,
                              add=True)
        h2 = pltpu.async_copy(ones_v, deg_sh.at[idx_v.at[j + 2]], sem,
                              add=True)
        h3 = pltpu.async_copy(ones_v, deg_sh.at[idx_v.at[j + 3]], sem,
                              add=True)
        h0.wait()
        h1.wait()
        h2.wait()
        h3.wait()

    plsc.subcore_barrier()
    pltpu.sync_copy(deg_sh.at[pl.ds(base, ROWS_PER_SUB)],
                    out_hbm.at[c].at[pl.ds(base, ROWS_PER_SUB)])

    @pl.when(s == NS - 1)
    def _():
        pltpu.sync_copy(deg_sh.at[pl.ds(NS * ROWS_PER_SUB, TAIL_ROWS)],
                        out_hbm.at[c].at[pl.ds(NS * ROWS_PER_SUB, TAIL_ROWS)])


@functools.partial(
    pl.kernel,
    out_type=jax.ShapeDtypeStruct((NC, N_PAD, D), jnp.float32),
    mesh=_mesh,
    scratch_types=[
        pltpu.VMEM((K // 2, CH), jnp.int32),
        pltpu.VMEM((K // 2, CH), jnp.int32),
        pltpu.VMEM((CH, D), jnp.float32),
        pltpu.VMEM((CH, D), jnp.float32),
        pltpu.VMEM_SHARED((N_PAD, D), jnp.float32),
        pltpu.SemaphoreType.DMA,
        pltpu.SemaphoreType.DMA,
    ],
)
def _agg_kernel(y_hbm, src_hbm, dst_hbm, zeros_hbm, out_hbm,
                src_v, dst_v, buf0, buf1, agg_sh, sem_g, sem_s):
    c = lax.axis_index("c")
    s = lax.axis_index("s")
    wid = s * NC + c
    base = s * ROWS_PER_SUB
    pltpu.sync_copy(zeros_hbm.at[pl.ds(base, ROWS_PER_SUB)],
                    agg_sh.at[pl.ds(base, ROWS_PER_SUB)])

    @pl.when(s == NS - 1)
    def _():
        pltpu.sync_copy(zeros_hbm.at[pl.ds(NS * ROWS_PER_SUB, TAIL_ROWS)],
                        agg_sh.at[pl.ds(NS * ROWS_PER_SUB, TAIL_ROWS)])

    plsc.subcore_barrier()

    @pl.loop(0, 2)
    def _(half):
        pltpu.sync_copy(src_hbm.at[wid].at[pl.ds(half * (K // 2), K // 2)],
                        src_v)
        pltpu.sync_copy(dst_hbm.at[wid].at[pl.ds(half * (K // 2), K // 2)],
                        dst_v)

        @pl.loop(0, K // 2, step=2)
        def _(j):
            g0 = pltpu.async_copy(y_hbm.at[src_v.at[j]], buf0, sem_g)
            g1 = pltpu.async_copy(y_hbm.at[src_v.at[j + 1]], buf1, sem_g)
            g0.wait()
            s0 = pltpu.async_copy(buf0, agg_sh.at[dst_v.at[j]], sem_s,
                                  add=True)
            g1.wait()
            s1 = pltpu.async_copy(buf1, agg_sh.at[dst_v.at[j + 1]], sem_s,
                                  add=True)
            s0.wait()
            s1.wait()

    plsc.subcore_barrier()
    pltpu.sync_copy(agg_sh.at[pl.ds(base, ROWS_PER_SUB)],
                    out_hbm.at[c].at[pl.ds(base, ROWS_PER_SUB)])

    @pl.when(s == NS - 1)
    def _():
        pltpu.sync_copy(agg_sh.at[pl.ds(NS * ROWS_PER_SUB, TAIL_ROWS)],
                        out_hbm.at[c].at[pl.ds(NS * ROWS_PER_SUB, TAIL_ROWS)])


def _scale_body(x_ref, degp_ref, y_ref):
    deg = degp_ref[0, :, 0] + degp_ref[1, :, 0]
    norm = 1.0 / jnp.maximum(deg, 1.0)
    y_ref[...] = x_ref[...] * norm[:, None]


def _dense_body(x_ref, aggp_ref, wt_ref, b_ref, o_ref):
    agg = aggp_ref[0] + aggp_ref[1]
    h = jnp.concatenate([x_ref[...], agg], axis=1)
    acc = jnp.dot(h, wt_ref[...], preferred_element_type=jnp.float32)
    o_ref[...] = jnp.maximum(acc + b_ref[...], 0.0)


_SCALE_BLK = 1000
_DENSE_BLK = 1000


def kernel(x, edge_index, W, b):
    src = edge_index[0].astype(jnp.int32)
    dst = edge_index[1].astype(jnp.int32)
    pad = N_NODES + (jnp.arange(E_PAD - N_EDGES, dtype=jnp.int32) % N_DUMP)
    src3 = jnp.concatenate([src, pad]).reshape(NW, K, CH)
    dst3 = jnp.concatenate([dst, pad]).reshape(NW, K, CH)
    srcd = src3.reshape(NW, DK, DCH)
    x_pad = jnp.pad(x, ((0, N_PAD - N_NODES), (0, 0)))
    onesd = jnp.ones((DCH, D), jnp.float32)
    zeros128 = jnp.zeros((N_PAD, D), jnp.float32)
    wt = W.T
    b2 = b.reshape(1, D)

    degp = _deg_kernel(srcd, zeros128, onesd)

    y = pl.pallas_call(
        _scale_body,
        grid=(N_PAD // _SCALE_BLK,),
        in_specs=[
            pl.BlockSpec((_SCALE_BLK, D), lambda i: (i, 0)),
            pl.BlockSpec((NC, _SCALE_BLK, D), lambda i: (0, i, 0)),
        ],
        out_specs=pl.BlockSpec((_SCALE_BLK, D), lambda i: (i, 0)),
        out_shape=jax.ShapeDtypeStruct((N_PAD, D), jnp.float32),
    )(x_pad, degp)

    aggp = _agg_kernel(y, src3, dst3, zeros128)

    out = pl.pallas_call(
        _dense_body,
        grid=(N_NODES // _DENSE_BLK,),
        in_specs=[
            pl.BlockSpec((_DENSE_BLK, D), lambda i: (i, 0)),
            pl.BlockSpec((NC, _DENSE_BLK, D), lambda i: (0, i, 0)),
            pl.BlockSpec((2 * D, D), lambda i: (0, 0)),
            pl.BlockSpec((1, D), lambda i: (0, 0)),
        ],
        out_specs=pl.BlockSpec((_DENSE_BLK, D), lambda i: (i, 0)),
        out_shape=jax.ShapeDtypeStruct((N_NODES, D), jnp.float32),
    )(x_pad, aggp, wt, b2)

    return out


# speedup vs baseline: 1.0299x; 1.0299x over previous
import functools

import jax
import jax.numpy as jnp
from jax import lax
from jax.experimental import pallas as pl
from jax.experimental.pallas import tpu as pltpu
from jax.experimental.pallas import tpu_sc as plsc

N_NODES = 10000
N_EDGES = 320000
D = 128

NC = 2
NS = 16
NW = NC * NS
CH = 128
K = 80
DCH = 128
DK = 80
E_PAD = NW * K * CH
N_DUMP = 1000
N_PAD = N_NODES + N_DUMP

ROWS_PER_SUB = (N_PAD // NS) // 8 * 8
TAIL_ROWS = N_PAD - NS * ROWS_PER_SUB

_mesh = plsc.VectorSubcoreMesh(core_axis_name="c", subcore_axis_name="s")


@functools.partial(
    pl.kernel,
    out_type=jax.ShapeDtypeStruct((NC, N_PAD, D), jnp.float32),
    mesh=_mesh,
    scratch_types=[
        pltpu.VMEM((DK, DCH), jnp.int32),
        pltpu.VMEM((DCH, D), jnp.float32),
        pltpu.VMEM_SHARED((N_PAD, D), jnp.float32),
        pltpu.SemaphoreType.DMA,
    ],
)
def _deg_kernel(src_hbm, zeros_hbm, ones_hbm, out_hbm, idx_v, ones_v, deg_sh,
                sem):
    c = lax.axis_index("c")
    s = lax.axis_index("s")
    wid = s * NC + c
    base = s * ROWS_PER_SUB
    pltpu.sync_copy(zeros_hbm.at[pl.ds(base, ROWS_PER_SUB)],
                    deg_sh.at[pl.ds(base, ROWS_PER_SUB)])

    @pl.when(s == NS - 1)
    def _():
        pltpu.sync_copy(zeros_hbm.at[pl.ds(NS * ROWS_PER_SUB, TAIL_ROWS)],
                        deg_sh.at[pl.ds(NS * ROWS_PER_SUB, TAIL_ROWS)])

    pltpu.sync_copy(ones_hbm, ones_v)
    pltpu.sync_copy(src_hbm.at[wid], idx_v)
    plsc.subcore_barrier()

    @pl.loop(0, DK, step=4)
    def _(j):
        h0 = pltpu.async_copy(ones_v, deg_sh.at[idx_v.at[j]], sem, add=True)
        h1 = pltpu.async_copy(ones_v, deg_sh.at[idx_v.at[j + 1]], sem,
                              add=True)
        h2 = pltpu.async_copy(ones_v, deg_sh.at[idx_v.at[j + 2]], sem,
                              add=True)
        h3 = pltpu.async_copy(ones_v, deg_sh.at[idx_v.at[j + 3]], sem,
                              add=True)
        h0.wait()
        h1.wait()
        h2.wait()
        h3.wait()

    plsc.subcore_barrier()
    pltpu.sync_copy(deg_sh.at[pl.ds(base, ROWS_PER_SUB)],
                    out_hbm.at[c].at[pl.ds(base, ROWS_PER_SUB)])

    @pl.when(s == NS - 1)
    def _():
        pltpu.sync_copy(deg_sh.at[pl.ds(NS * ROWS_PER_SUB, TAIL_ROWS)],
                        out_hbm.at[c].at[pl.ds(NS * ROWS_PER_SUB, TAIL_ROWS)])


@functools.partial(
    pl.kernel,
    out_type=jax.ShapeDtypeStruct((NC, N_PAD, D), jnp.float32),
    mesh=_mesh,
    scratch_types=[
        pltpu.VMEM((K // 2, CH), jnp.int32),
        pltpu.VMEM((K // 2, CH), jnp.int32),
        pltpu.VMEM((CH // 2, D), jnp.float32),
        pltpu.VMEM((CH // 2, D), jnp.float32),
        pltpu.VMEM((CH // 2, D), jnp.float32),
        pltpu.VMEM((CH // 2, D), jnp.float32),
        pltpu.VMEM_SHARED((N_PAD, D), jnp.float32),
        pltpu.SemaphoreType.DMA,
        pltpu.SemaphoreType.DMA,
    ],
)
def _agg_kernel(y_hbm, src_hbm, dst_hbm, zeros_hbm, out_hbm,
                src_v, dst_v, buf0, buf1, buf2, buf3, agg_sh, sem_g, sem_s):
    c = lax.axis_index("c")
    s = lax.axis_index("s")
    wid = s * NC + c
    base = s * ROWS_PER_SUB
    pltpu.sync_copy(zeros_hbm.at[pl.ds(base, ROWS_PER_SUB)],
                    agg_sh.at[pl.ds(base, ROWS_PER_SUB)])

    @pl.when(s == NS - 1)
    def _():
        pltpu.sync_copy(zeros_hbm.at[pl.ds(NS * ROWS_PER_SUB, TAIL_ROWS)],
                        agg_sh.at[pl.ds(NS * ROWS_PER_SUB, TAIL_ROWS)])

    plsc.subcore_barrier()

    @pl.loop(0, 2)
    def _(half):
        pltpu.sync_copy(src_hbm.at[wid].at[pl.ds(half * (K // 2), K // 2)],
                        src_v)
        pltpu.sync_copy(dst_hbm.at[wid].at[pl.ds(half * (K // 2), K // 2)],
                        dst_v)

        H = CH // 2

        @pl.loop(0, K // 2, step=2)
        def _(j):
            g0 = pltpu.async_copy(y_hbm.at[src_v.at[j, pl.ds(0, H)]],
                                  buf0, sem_g)
            g1 = pltpu.async_copy(y_hbm.at[src_v.at[j, pl.ds(H, H)]],
                                  buf1, sem_g)
            g2 = pltpu.async_copy(y_hbm.at[src_v.at[j + 1, pl.ds(0, H)]],
                                  buf2, sem_g)
            g3 = pltpu.async_copy(y_hbm.at[src_v.at[j + 1, pl.ds(H, H)]],
                                  buf3, sem_g)
            g0.wait()
            s0 = pltpu.async_copy(buf0, agg_sh.at[dst_v.at[j, pl.ds(0, H)]],
                                  sem_s, add=True)
            g1.wait()
            s1 = pltpu.async_copy(buf1, agg_sh.at[dst_v.at[j, pl.ds(H, H)]],
                                  sem_s, add=True)
            g2.wait()
            s2 = pltpu.async_copy(buf2,
                                  agg_sh.at[dst_v.at[j + 1, pl.ds(0, H)]],
                                  sem_s, add=True)
            g3.wait()
            s3 = pltpu.async_copy(buf3,
                                  agg_sh.at[dst_v.at[j + 1, pl.ds(H, H)]],
                                  sem_s, add=True)
            s0.wait()
            s1.wait()
            s2.wait()
            s3.wait()

    plsc.subcore_barrier()
    pltpu.sync_copy(agg_sh.at[pl.ds(base, ROWS_PER_SUB)],
                    out_hbm.at[c].at[pl.ds(base, ROWS_PER_SUB)])

    @pl.when(s == NS - 1)
    def _():
        pltpu.sync_copy(agg_sh.at[pl.ds(NS * ROWS_PER_SUB, TAIL_ROWS)],
                        out_hbm.at[c].at[pl.ds(NS * ROWS_PER_SUB, TAIL_ROWS)])


def _scale_body(x_ref, degp_ref, y_ref):
    deg = degp_ref[0, :, 0] + degp_ref[1, :, 0]
    norm = 1.0 / jnp.maximum(deg, 1.0)
    y_ref[...] = x_ref[...] * norm[:, None]


def _dense_body(x_ref, aggp_ref, wt_ref, b_ref, o_ref):
    agg = aggp_ref[0] + aggp_ref[1]
    h = jnp.concatenate([x_ref[...], agg], axis=1)
    acc = jnp.dot(h, wt_ref[...], preferred_element_type=jnp.float32)
    o_ref[...] = jnp.maximum(acc + b_ref[...], 0.0)


_SCALE_BLK = 1000
_DENSE_BLK = 1000


def kernel(x, edge_index, W, b):
    src = edge_index[0].astype(jnp.int32)
    dst = edge_index[1].astype(jnp.int32)
    pad = N_NODES + (jnp.arange(E_PAD - N_EDGES, dtype=jnp.int32) % N_DUMP)
    src3 = jnp.concatenate([src, pad]).reshape(NW, K, CH)
    dst3 = jnp.concatenate([dst, pad]).reshape(NW, K, CH)
    srcd = src3.reshape(NW, DK, DCH)
    x_pad = jnp.pad(x, ((0, N_PAD - N_NODES), (0, 0)))
    onesd = jnp.ones((DCH, D), jnp.float32)
    zeros128 = jnp.zeros((N_PAD, D), jnp.float32)
    wt = W.T
    b2 = b.reshape(1, D)

    degp = _deg_kernel(srcd, zeros128, onesd)

    y = pl.pallas_call(
        _scale_body,
        grid=(N_PAD // _SCALE_BLK,),
        in_specs=[
            pl.BlockSpec((_SCALE_BLK, D), lambda i: (i, 0)),
            pl.BlockSpec((NC, _SCALE_BLK, D), lambda i: (0, i, 0)),
        ],
        out_specs=pl.BlockSpec((_SCALE_BLK, D), lambda i: (i, 0)),
        out_shape=jax.ShapeDtypeStruct((N_PAD, D), jnp.float32),
    )(x_pad, degp)

    aggp = _agg_kernel(y, src3, dst3, zeros128)

    out = pl.pallas_call(
        _dense_body,
        grid=(N_NODES // _DENSE_BLK,),
        in_specs=[
            pl.BlockSpec((_DENSE_BLK, D), lambda i: (i, 0)),
            pl.BlockSpec((NC, _DENSE_BLK, D), lambda i: (0, i, 0)),
            pl.BlockSpec((2 * D, D), lambda i: (0, 0)),
            pl.BlockSpec((1, D), lambda i: (0, 0)),
        ],
        out_specs=pl.BlockSpec((_DENSE_BLK, D), lambda i: (i, 0)),
        out_shape=jax.ShapeDtypeStruct((N_NODES, D), jnp.float32),
    )(x_pad, aggp, wt, b2)

    return out
